# Initial kernel scaffold; baseline (speedup 1.0000x reference)
import jax
import jax.numpy as jnp
from jax.experimental import pallas as pl


def kernel(item, nbr_mask, user_id, item_input_lookup, user_embedding_matrix):
    raise NotImplementedError("write your pallas kernel here")


# speedup vs baseline: 1.4247x; 1.4247x over previous
import functools

import jax
import jax.numpy as jnp
from jax import lax
from jax.experimental import pallas as pl
from jax.experimental.pallas import tpu as pltpu
from jax.experimental.pallas import tpu_sc as plsc

N_MID = 1000000
USER_COUNT = 100000
DIM = 32
B = 4096
SEQ = 200
BSEQ = B * SEQ

NC = 2
NS = 16
NW = NC * NS

PER_W = BSEQ // NW
STREAM = 128
K = 8
CHUNK = K * STREAM
NCHUNK = PER_W // CHUNK
U_PER_W = B // NW

_mesh = plsc.VectorSubcoreMesh(
    core_axis_name="c", subcore_axis_name="s", num_cores=NC, num_subcores=NS
)


@functools.partial(
    pl.kernel,
    out_type=(
        jax.ShapeDtypeStruct((BSEQ, DIM), jnp.float32),
        jax.ShapeDtypeStruct((B, DIM), jnp.float32),
    ),
    mesh=_mesh,
    scratch_types=[
        pltpu.VMEM((CHUNK,), jnp.int32),
        pltpu.VMEM((CHUNK, DIM), jnp.float32),
        pltpu.VMEM((U_PER_W,), jnp.int32),
        pltpu.VMEM((U_PER_W, DIM), jnp.float32),
        pltpu.SemaphoreType.DMA,
    ],
    compiler_params=pltpu.CompilerParams(use_tc_tiling_on_sc=False),
)
def _sc_gather(
    item_hbm, user_hbm, table_hbm, utable_hbm,
    item_out, user_out,
    idx_v, rows_v, uidx_v, urows_v, sem,
):
    wid = lax.axis_index("s") * NC + lax.axis_index("c")

    ubase = wid * U_PER_W
    pltpu.sync_copy(user_hbm.at[pl.ds(ubase, U_PER_W)], uidx_v)
    pltpu.async_copy(utable_hbm.at[uidx_v], urows_v, sem).wait()
    pltpu.sync_copy(urows_v, user_out.at[pl.ds(ubase, U_PER_W)])

    base0 = wid * PER_W

    @pl.loop(0, NCHUNK)
    def _chunk(t):
        base = base0 + t * CHUNK
        pltpu.sync_copy(item_hbm.at[pl.ds(base, CHUNK)], idx_v)
        copies = []
        for j in range(K):
            copies.append(
                pltpu.async_copy(
                    table_hbm.at[idx_v.at[pl.ds(j * STREAM, STREAM)]],
                    rows_v.at[pl.ds(j * STREAM, STREAM)],
                    sem,
                )
            )
        for c in copies:
            c.wait()
        pltpu.sync_copy(rows_v, item_out.at[pl.ds(base, CHUNK)])


def _mask_body(mask_ref, out_ref):
    out_ref[...] = jnp.sum(mask_ref[...], axis=1).astype(jnp.int32)


def kernel(item, nbr_mask, user_id, item_input_lookup, user_embedding_matrix):
    item_flat = item.reshape(-1)
    item_emb_flat, user_embedding = _sc_gather(
        item_flat, user_id, item_input_lookup, user_embedding_matrix
    )
    item_emb = item_emb_flat.reshape(B, SEQ, DIM)
    mask_length = pl.pallas_call(
        _mask_body,
        out_shape=jax.ShapeDtypeStruct((B,), jnp.int32),
    )(nbr_mask)
    return item_emb, user_embedding, mask_length


# speedup vs baseline: 1.4556x; 1.0216x over previous
import functools

import jax
import jax.numpy as jnp
from jax import lax
from jax.experimental import pallas as pl
from jax.experimental.pallas import tpu as pltpu
from jax.experimental.pallas import tpu_sc as plsc

N_MID = 1000000
USER_COUNT = 100000
DIM = 32
B = 4096
SEQ = 200
BSEQ = B * SEQ

NC = 2
NS = 16
NW = NC * NS

PER_W = BSEQ // NW
STREAM = 128
K = 5
CHUNK = K * STREAM
NCHUNK = PER_W // CHUNK
NPAIR = NCHUNK // 2
U_PER_W = B // NW

_mesh = plsc.VectorSubcoreMesh(
    core_axis_name="c", subcore_axis_name="s", num_cores=NC, num_subcores=NS
)


@functools.partial(
    pl.kernel,
    out_type=(
        jax.ShapeDtypeStruct((BSEQ, DIM), jnp.float32),
        jax.ShapeDtypeStruct((B, DIM), jnp.float32),
    ),
    mesh=_mesh,
    scratch_types=[
        pltpu.VMEM((PER_W,), jnp.int32),
        pltpu.VMEM((CHUNK, DIM), jnp.float32),
        pltpu.VMEM((CHUNK, DIM), jnp.float32),
        pltpu.VMEM((U_PER_W,), jnp.int32),
        pltpu.VMEM((U_PER_W, DIM), jnp.float32),
        pltpu.SemaphoreType.DMA,
        pltpu.SemaphoreType.DMA,
        pltpu.SemaphoreType.DMA,
        pltpu.SemaphoreType.DMA,
        pltpu.SemaphoreType.DMA,
    ],
    compiler_params=pltpu.CompilerParams(use_tc_tiling_on_sc=False),
)
def _sc_gather(
    item_hbm, user_hbm, table_hbm, utable_hbm,
    item_out, user_out,
    idx_all, rows0, rows1, uidx_v, urows_v,
    g0, g1, w0, w1, usem,
):
    wid = lax.axis_index("s") * NC + lax.axis_index("c")
    base0 = wid * PER_W
    rows = (rows0, rows1)
    gsem = (g0, g1)

    ubase = wid * U_PER_W
    pltpu.sync_copy(user_hbm.at[pl.ds(ubase, U_PER_W)], uidx_v)
    ucopy = pltpu.async_copy(utable_hbm.at[uidx_v], urows_v, usem)

    pltpu.sync_copy(item_hbm.at[pl.ds(base0, PER_W)], idx_all)

    def fire(t, b):
        off = t * CHUNK
        for j in range(K):
            pltpu.async_copy(
                table_hbm.at[idx_all.at[pl.ds(off + j * STREAM, STREAM)]],
                rows[b].at[pl.ds(j * STREAM, STREAM)],
                gsem[b],
            )

    def drain(b):
        pltpu.make_async_copy(
            table_hbm.at[pl.ds(0, CHUNK)], rows[b], gsem[b]
        ).wait()

    def writeback(t, b, sem):
        return pltpu.async_copy(
            rows[b], item_out.at[pl.ds(base0 + t * CHUNK, CHUNK)], sem
        )

    fire(0, 0)
    fire(1, 1)

    @pl.loop(0, NPAIR - 1)
    def _pair(i):
        t0 = 2 * i
        drain(0)
        wb0 = writeback(t0, 0, w0)
        drain(1)
        wb1 = writeback(t0 + 1, 1, w1)
        wb0.wait()
        fire(t0 + 2, 0)
        wb1.wait()
        fire(t0 + 3, 1)

    drain(0)
    wb0 = writeback(NCHUNK - 2, 0, w0)
    drain(1)
    wb1 = writeback(NCHUNK - 1, 1, w1)
    wb0.wait()
    wb1.wait()

    ucopy.wait()
    pltpu.sync_copy(urows_v, user_out.at[pl.ds(ubase, U_PER_W)])


def _mask_body(mask_ref, out_ref):
    out_ref[...] = jnp.sum(mask_ref[...], axis=1).astype(jnp.int32)


def kernel(item, nbr_mask, user_id, item_input_lookup, user_embedding_matrix):
    item_flat = item.reshape(-1)
    item_emb_flat, user_embedding = _sc_gather(
        item_flat, user_id, item_input_lookup, user_embedding_matrix
    )
    item_emb = item_emb_flat.reshape(B, SEQ, DIM)
    mask_length = pl.pallas_call(
        _mask_body,
        out_shape=jax.ShapeDtypeStruct((B,), jnp.int32),
    )(nbr_mask)
    return item_emb, user_embedding, mask_length
